# Initial kernel scaffold; baseline (speedup 1.0000x reference)
import jax
import jax.numpy as jnp
from jax.experimental import pallas as pl


def kernel(category_indices, table, norms_table):
    raise NotImplementedError("write your pallas kernel here")


# speedup vs baseline: 5.2595x; 5.2595x over previous
import functools

import jax
import jax.numpy as jnp
from jax import lax
from jax.experimental import pallas as pl
from jax.experimental.pallas import tpu as pltpu
from jax.experimental.pallas import tpu_sc as plsc

B = 16384
D = 128
V = 1000
L = 16
NC = 2
NS = 16
NW = NC * NS
BPW = B // NW
RPT = 64
IDX_CHUNK = 128

_mesh = plsc.VectorSubcoreMesh(core_axis_name="c", subcore_axis_name="s")


@functools.partial(
    pl.kernel,
    mesh=_mesh,
    out_type=jax.ShapeDtypeStruct((B, D), jnp.float32),
    compiler_params=pltpu.CompilerParams(needs_layout_passes=False),
    scratch_types=[
        pltpu.VMEM((RPT, D), jnp.float32),
        pltpu.VMEM((RPT,), jnp.float32),
        pltpu.VMEM((BPW,), jnp.int32),
        pltpu.VMEM((BPW, D), jnp.float32),
        pltpu.VMEM_SHARED((1024, D), jnp.float32),
        pltpu.SemaphoreType.DMA,
    ],
)
def _emb_clip(idx_hbm, table_hbm, norms_hbm, out_hbm,
              chunk_v, norms_v, idx_v, rows_v, shared_tab, sem):
    cid = lax.axis_index("c")
    sid = lax.axis_index("s")

    start = jnp.minimum(sid * RPT, V - RPT)
    pltpu.sync_copy(table_hbm.at[pl.ds(start, RPT)], chunk_v)
    pltpu.sync_copy(norms_hbm.at[pl.ds(start, RPT)], norms_v)

    row0 = lax.iota(jnp.int32, L)
    for g in range(RPT // L):
        rows16 = row0 + (g * L)
        acc = jnp.zeros((L,), jnp.float32)

        def col_body(c, acc, rows16=rows16):
            for u in range(8):
                col = plsc.load_gather(
                    chunk_v, [rows16, jnp.full((L,), c * 8 + u, jnp.int32)])
                acc = acc + col * col
            return acc

        sumsq = lax.fori_loop(0, D // 8, col_body, acc, unroll=False)
        x = jnp.maximum(sumsq, jnp.float32(1e-16))
        bits = plsc.bitcast(x, jnp.int32)
        bits = jnp.int32(0x5F3759DF) - (bits >> 1)
        y = plsc.bitcast(bits, jnp.float32)
        for _ in range(3):
            y = y * (jnp.float32(1.5) - jnp.float32(0.5) * x * y * y)
        cur = x * y
        n16 = norms_v[pl.ds(g * L, L)]
        scale16 = jnp.minimum(jnp.float32(1.0), n16 / cur)

        for i in range(L):
            r = g * L + i
            sv = jnp.full((L,), scale16[i], jnp.float32)
            for j in range(D // L):
                chunk_v[r, pl.ds(j * L, L)] = chunk_v[r, pl.ds(j * L, L)] * sv
    pltpu.sync_copy(chunk_v, shared_tab.at[pl.ds(start, RPT)])
    plsc.subcore_barrier()

    wid = sid * NC + cid
    base = wid * BPW
    pltpu.sync_copy(idx_hbm.at[pl.ds(base, BPW)], idx_v)
    copies = []
    for j in range(BPW // IDX_CHUNK):
        copies.append(pltpu.async_copy(
            shared_tab.at[idx_v.at[pl.ds(j * IDX_CHUNK, IDX_CHUNK)]],
            rows_v.at[pl.ds(j * IDX_CHUNK, IDX_CHUNK)],
            sem))
    for cp in copies:
        cp.wait()
    pltpu.sync_copy(rows_v, out_hbm.at[pl.ds(base, BPW)])


def kernel(category_indices, table, norms_table):
    return _emb_clip(category_indices, table, norms_table)
